# baseline (speedup 1.0000x reference)
import functools

import jax
import jax.numpy as jnp
from jax import lax
from jax.experimental import pallas as pl
from jax.experimental.pallas import tpu as pltpu
from jax.experimental.pallas import tpu_sc as plsc

N, C, H, W = 4, 96, 384, 384
HW = H * W
NP = N * HW
NC, NS, L = 2, 16, 16
NW = NC * NS
PXW = NP // NW
P = 128
CHUNKS = PXW // P


def _sc_grid_sample(xt, gx, gy):
    mesh = plsc.VectorSubcoreMesh(
        core_axis_name="c", subcore_axis_name="s", num_cores=NC,
        num_subcores=NS)

    @functools.partial(
        pl.kernel,
        out_type=jax.ShapeDtypeStruct((N * C, HW), jnp.float32),
        mesh=mesh,
        scratch_types=[
            pltpu.VMEM((P,), jnp.float32),
            pltpu.VMEM((P,), jnp.float32),
            pltpu.VMEM((P,), jnp.int32),
            pltpu.VMEM((P,), jnp.int32),
            pltpu.VMEM((P,), jnp.int32),
            pltpu.VMEM((P,), jnp.int32),
            pltpu.VMEM((P,), jnp.float32),
            pltpu.VMEM((P,), jnp.float32),
            pltpu.VMEM((P,), jnp.float32),
            pltpu.VMEM((P,), jnp.float32),
            pltpu.VMEM((P, C), jnp.float32),
            pltpu.VMEM((P, C), jnp.float32),
            pltpu.VMEM((P, C), jnp.float32),
            pltpu.VMEM((P, C), jnp.float32),
            pltpu.VMEM((C, P), jnp.float32),
            pltpu.SemaphoreType.DMA,
        ],
        compiler_params=pltpu.CompilerParams(
            use_tc_tiling_on_sc=False, needs_layout_passes=False),
    )
    def k(xt_hbm, gx_hbm, gy_hbm, out_hbm,
          gx_v, gy_v, i00_v, i01_v, i10_v, i11_v,
          w00_v, w01_v, w10_v, w11_v,
          r00_v, r01_v, r10_v, r11_v, ob_v, sem):
        wid = lax.axis_index("s") * NC + lax.axis_index("c")
        px_base = wid * PXW
        batch = px_base // HW
        batch_off = batch * HW
        hw_base = px_base - batch_off

        def chunk_body(g, carry):
            base = px_base + g * P
            pltpu.sync_copy(gx_hbm.at[pl.ds(base, P)], gx_v)
            pltpu.sync_copy(gy_hbm.at[pl.ds(base, P)], gy_v)

            for v in range(P // L):
                sl = pl.ds(v * L, L)
                gxv = gx_v[sl]
                gyv = gy_v[sl]
                ix = (gxv + 1.0) * (W * 0.5) - 0.5
                iy = (gyv + 1.0) * (H * 0.5) - 0.5
                tx = ix.astype(jnp.int32).astype(jnp.float32)
                ix0f = jnp.where(tx > ix, tx - 1.0, tx)
                ty = iy.astype(jnp.int32).astype(jnp.float32)
                iy0f = jnp.where(ty > iy, ty - 1.0, ty)
                wx1 = ix - ix0f
                wx0 = 1.0 - wx1
                wy1 = iy - iy0f
                wy0 = 1.0 - wy1
                ix0 = ix0f.astype(jnp.int32)
                ix1 = ix0 + 1
                iy0 = iy0f.astype(jnp.int32)
                iy1 = iy0 + 1
                vx0 = jnp.where((ix0 >= 0) & (ix0 < W), 1.0, 0.0)
                vx1 = jnp.where((ix1 >= 0) & (ix1 < W), 1.0, 0.0)
                vy0 = jnp.where((iy0 >= 0) & (iy0 < H), 1.0, 0.0)
                vy1 = jnp.where((iy1 >= 0) & (iy1 < H), 1.0, 0.0)
                xc0 = jnp.minimum(jnp.maximum(ix0, 0), W - 1)
                xc1 = jnp.minimum(jnp.maximum(ix1, 0), W - 1)
                yc0 = jnp.minimum(jnp.maximum(iy0, 0), H - 1)
                yc1 = jnp.minimum(jnp.maximum(iy1, 0), H - 1)
                r0 = yc0 * W + batch_off
                r1 = yc1 * W + batch_off
                i00_v[sl] = r0 + xc0
                i01_v[sl] = r0 + xc1
                i10_v[sl] = r1 + xc0
                i11_v[sl] = r1 + xc1
                w00_v[sl] = wy0 * wx0 * vy0 * vx0
                w01_v[sl] = wy0 * wx1 * vy0 * vx1
                w10_v[sl] = wy1 * wx0 * vy1 * vx0
                w11_v[sl] = wy1 * wx1 * vy1 * vx1

            cps = [
                pltpu.async_copy(xt_hbm.at[i00_v], r00_v, sem),
                pltpu.async_copy(xt_hbm.at[i01_v], r01_v, sem),
                pltpu.async_copy(xt_hbm.at[i10_v], r10_v, sem),
                pltpu.async_copy(xt_hbm.at[i11_v], r11_v, sem),
            ]
            for cp in cps:
                cp.wait()

            citers = [lax.iota(jnp.int32, L) + j * L for j in range(C // L)]

            def grp_body(q, c2):
                qb = q * L
                sg = pl.ds(qb, L)
                wg00 = w00_v[sg]
                wg01 = w01_v[sg]
                wg10 = w10_v[sg]
                wg11 = w11_v[sg]
                for lane in range(L):
                    p = qb + lane
                    pv = lax.broadcast(p, (L,))
                    b00 = lax.broadcast(wg00[lane], (L,))
                    b01 = lax.broadcast(wg01[lane], (L,))
                    b10 = lax.broadcast(wg10[lane], (L,))
                    b11 = lax.broadcast(wg11[lane], (L,))
                    for j in range(C // L):
                        sj = pl.ds(j * L, L)
                        acc = (r00_v[p, sj] * b00 + r01_v[p, sj] * b01
                               + r10_v[p, sj] * b10 + r11_v[p, sj] * b11)
                        plsc.store_scatter(ob_v, [citers[j], pv], acc)
                return c2

            lax.fori_loop(0, P // L, grp_body, 0, unroll=False)
            pltpu.sync_copy(
                ob_v,
                out_hbm.at[pl.ds(batch * C, C), pl.ds(hw_base + g * P, P)])
            return carry

        lax.fori_loop(0, CHUNKS, chunk_body, 0, unroll=False)

    return k(xt, gx, gy)


def kernel(x, grid):
    xt = x.transpose(0, 2, 3, 1).reshape(NP, C)
    gx = grid[..., 0].reshape(NP)
    gy = grid[..., 1].reshape(NP)
    out = _sc_grid_sample(xt, gx, gy)
    return out.reshape(N, C, H, W)


# speedup vs baseline: 1.6828x; 1.6828x over previous
import functools

import jax
import jax.numpy as jnp
from jax import lax
from jax.experimental import pallas as pl
from jax.experimental.pallas import tpu as pltpu
from jax.experimental.pallas import tpu_sc as plsc

N, C, H, W = 4, 96, 384, 384
HW = H * W
NP = N * HW
NC, NS, L = 2, 16, 16
NW = NC * NS
PXW = NP // NW
P = 128
CHUNKS = PXW // P


def _sc_grid_sample(xt, gxy):
    mesh = plsc.VectorSubcoreMesh(
        core_axis_name="c", subcore_axis_name="s", num_cores=NC,
        num_subcores=NS)

    scratch = (
        [pltpu.VMEM((2, P), jnp.float32)]
        + [pltpu.VMEM((P,), jnp.int32)] * 8
        + [pltpu.VMEM((P,), jnp.float32)] * 8
        + [pltpu.VMEM((P, C), jnp.float32)] * 8
        + [pltpu.VMEM((P, C), jnp.float32)]
        + [pltpu.SemaphoreType.DMA] * 2
    )

    @functools.partial(
        pl.kernel,
        out_type=jax.ShapeDtypeStruct((NP, C), jnp.float32),
        mesh=mesh,
        scratch_types=scratch,
        compiler_params=pltpu.CompilerParams(use_tc_tiling_on_sc=False),
    )
    def k(xt_hbm, gxy_hbm, out_hbm, gxy_v, *rest):
        ii = [rest[0:4], rest[4:8]]
        ww = [rest[8:12], rest[12:16]]
        rr = [rest[16:20], rest[20:24]]
        ob_v = rest[24]
        sems = [rest[25], rest[26]]

        wid = lax.axis_index("s") * NC + lax.axis_index("c")
        px_base = wid * PXW
        batch_off = (px_base // HW) * HW

        def stage(g, s):
            base = px_base + g * P
            pltpu.sync_copy(gxy_hbm.at[:, pl.ds(base, P)], gxy_v)
            for v in range(P // L):
                sl = pl.ds(v * L, L)
                gxv = gxy_v[0, sl]
                gyv = gxy_v[1, sl]
                ix = (gxv + 1.0) * (W * 0.5) - 0.5
                iy = (gyv + 1.0) * (H * 0.5) - 0.5
                tx = ix.astype(jnp.int32).astype(jnp.float32)
                ix0f = jnp.where(tx > ix, tx - 1.0, tx)
                ty = iy.astype(jnp.int32).astype(jnp.float32)
                iy0f = jnp.where(ty > iy, ty - 1.0, ty)
                wx1 = ix - ix0f
                wx0 = 1.0 - wx1
                wy1 = iy - iy0f
                wy0 = 1.0 - wy1
                ix0 = ix0f.astype(jnp.int32)
                ix1 = ix0 + 1
                iy0 = iy0f.astype(jnp.int32)
                iy1 = iy0 + 1
                vx0 = jnp.where((ix0 >= 0) & (ix0 < W), 1.0, 0.0)
                vx1 = jnp.where((ix1 >= 0) & (ix1 < W), 1.0, 0.0)
                vy0 = jnp.where((iy0 >= 0) & (iy0 < H), 1.0, 0.0)
                vy1 = jnp.where((iy1 >= 0) & (iy1 < H), 1.0, 0.0)
                xc0 = jnp.minimum(jnp.maximum(ix0, 0), W - 1)
                xc1 = jnp.minimum(jnp.maximum(ix1, 0), W - 1)
                yc0 = jnp.minimum(jnp.maximum(iy0, 0), H - 1)
                yc1 = jnp.minimum(jnp.maximum(iy1, 0), H - 1)
                r0 = yc0 * W + batch_off
                r1 = yc1 * W + batch_off
                ii[s][0][sl] = r0 + xc0
                ii[s][1][sl] = r0 + xc1
                ii[s][2][sl] = r1 + xc0
                ii[s][3][sl] = r1 + xc1
                ww[s][0][sl] = wy0 * wx0 * vy0 * vx0
                ww[s][1][sl] = wy0 * wx1 * vy0 * vx1
                ww[s][2][sl] = wy1 * wx0 * vy1 * vx0
                ww[s][3][sl] = wy1 * wx1 * vy1 * vx1
            for c in range(4):
                pltpu.async_copy(xt_hbm.at[ii[s][c]], rr[s][c], sems[s])

        def finish(g, s):
            base = px_base + g * P
            for c in range(4):
                pltpu.make_async_copy(
                    xt_hbm.at[ii[s][c]], rr[s][c], sems[s]).wait()
            r00_v, r01_v, r10_v, r11_v = rr[s]

            def grp_body(q, c2):
                qb = q * L
                sg = pl.ds(qb, L)
                wg00 = ww[s][0][sg]
                wg01 = ww[s][1][sg]
                wg10 = ww[s][2][sg]
                wg11 = ww[s][3][sg]
                for lane in range(L):
                    p = qb + lane
                    b00 = lax.broadcast(wg00[lane], (L,))
                    b01 = lax.broadcast(wg01[lane], (L,))
                    b10 = lax.broadcast(wg10[lane], (L,))
                    b11 = lax.broadcast(wg11[lane], (L,))
                    for j in range(C // L):
                        sj = pl.ds(j * L, L)
                        ob_v[p, sj] = (
                            r00_v[p, sj] * b00 + r01_v[p, sj] * b01
                            + r10_v[p, sj] * b10 + r11_v[p, sj] * b11)
                return c2

            lax.fori_loop(0, P // L, grp_body, 0, unroll=False)
            pltpu.sync_copy(ob_v, out_hbm.at[pl.ds(base, P)])

        stage(0, 0)

        def body(t, carry):
            g0 = 2 * t
            stage(g0 + 1, 1)
            finish(g0, 0)

            @pl.when(t < CHUNKS // 2 - 1)
            def _():
                stage(g0 + 2, 0)

            finish(g0 + 1, 1)
            return carry

        lax.fori_loop(0, CHUNKS // 2, body, 0, unroll=False)

    return k(xt, gxy)


def kernel(x, grid):
    xt = x.transpose(0, 2, 3, 1).reshape(NP, C)
    gxy = grid.reshape(NP, 2).transpose(1, 0)
    out = _sc_grid_sample(xt, gxy)
    return out.reshape(N, H, W, C).transpose(0, 3, 1, 2)
